# baseline (speedup 1.0000x reference)
import functools

import jax
import jax.numpy as jnp
from jax import lax
from jax.experimental import pallas as pl
from jax.experimental.pallas import tpu as pltpu
from jax.experimental.pallas import tpu_sc as plsc

TOTAL_TOKENS = 32768
BATCH = 16
D = 1024
BLOCK_ROWS = 2048
NSPLITS = BATCH + 1

_sc_mesh = plsc.VectorSubcoreMesh(core_axis_name="c", subcore_axis_name="s",
                                  num_cores=1, num_subcores=1)


@functools.partial(
    pl.kernel,
    mesh=_sc_mesh,
    out_type=jax.ShapeDtypeStruct((NSPLITS,), jnp.int32),
    scratch_types=[
        pltpu.VMEM((NSPLITS,), jnp.int32),
        pltpu.VMEM((NSPLITS,), jnp.int32),
    ],
)
def _sc_row_splits(rs_hbm, out_hbm, ibuf, obuf):
    pltpu.sync_copy(rs_hbm, ibuf)
    lanes = lax.iota(jnp.int32, 16)
    lo = ibuf[pl.ds(0, 16)]
    hi = ibuf[pl.ds(1, 16)]
    cond = hi >= lo
    cond = cond & ((lanes != 0) | (lo == 0))
    cond = cond & ((lanes != 15) | (hi == TOTAL_TOKENS))
    ok = plsc.all_reduce_population_count(cond) == 16
    obuf[pl.ds(0, 16)] = jnp.where(ok, lo, lo)
    obuf[pl.ds(1, 16)] = jnp.where(ok, hi, hi)
    pltpu.sync_copy(obuf, out_hbm)


NCHUNK = 8
CHUNK = TOTAL_TOKENS // NCHUNK


def _dma_copy_body(x_hbm, o_hbm, sems):
    for i in range(NCHUNK):
        pltpu.make_async_copy(
            x_hbm.at[pl.ds(i * CHUNK, CHUNK), :],
            o_hbm.at[pl.ds(i * CHUNK, CHUNK), :],
            sems.at[i],
        ).start()
    for i in range(NCHUNK):
        pltpu.make_async_copy(
            x_hbm.at[pl.ds(i * CHUNK, CHUNK), :],
            o_hbm.at[pl.ds(i * CHUNK, CHUNK), :],
            sems.at[i],
        ).wait()


def kernel(flat, row_splits):
    values = pl.pallas_call(
        _dma_copy_body,
        in_specs=[pl.BlockSpec(memory_space=pl.ANY)],
        out_specs=pl.BlockSpec(memory_space=pl.ANY),
        out_shape=jax.ShapeDtypeStruct((TOTAL_TOKENS, D), jnp.float32),
        scratch_shapes=[pltpu.SemaphoreType.DMA((NCHUNK,))],
    )(flat)
    rs_out = _sc_row_splits(row_splits)
    return (values, rs_out)


# speedup vs baseline: 38.2309x; 38.2309x over previous
import functools

import jax
import jax.numpy as jnp
from jax import lax
from jax.experimental import pallas as pl
from jax.experimental.pallas import tpu as pltpu
from jax.experimental.pallas import tpu_sc as plsc

TOTAL_TOKENS = 32768
BATCH = 16
D = 1024
BLOCK_ROWS = 512
NSPLITS = BATCH + 1

_sc_mesh = plsc.VectorSubcoreMesh(core_axis_name="c", subcore_axis_name="s",
                                  num_cores=1, num_subcores=1)


@functools.partial(
    pl.kernel,
    mesh=_sc_mesh,
    out_type=jax.ShapeDtypeStruct((NSPLITS,), jnp.int32),
    scratch_types=[
        pltpu.VMEM((NSPLITS,), jnp.int32),
        pltpu.VMEM((NSPLITS,), jnp.int32),
    ],
)
def _sc_row_splits(rs_hbm, out_hbm, ibuf, obuf):
    pltpu.sync_copy(rs_hbm, ibuf)
    lanes = lax.iota(jnp.int32, 16)
    lo = ibuf[pl.ds(0, 16)]
    hi = ibuf[pl.ds(1, 16)]
    cond = hi >= lo
    cond = cond & ((lanes != 0) | (lo == 0))
    cond = cond & ((lanes != 15) | (hi == TOTAL_TOKENS))
    ok = plsc.all_reduce_population_count(cond) == 16
    obuf[pl.ds(0, 16)] = jnp.where(ok, lo, lo)
    obuf[pl.ds(1, 16)] = jnp.where(ok, hi, hi)
    pltpu.sync_copy(obuf, out_hbm)


def _copy_body(x_ref, o_ref):
    o_ref[...] = x_ref[...]


def kernel(flat, row_splits):
    values = pl.pallas_call(
        _copy_body,
        grid=(TOTAL_TOKENS // BLOCK_ROWS,),
        in_specs=[pl.BlockSpec((BLOCK_ROWS, D), lambda i: (i, 0))],
        out_specs=pl.BlockSpec((BLOCK_ROWS, D), lambda i: (i, 0)),
        out_shape=jax.ShapeDtypeStruct((TOTAL_TOKENS, D), jnp.float32),
        compiler_params=pltpu.CompilerParams(
            dimension_semantics=("parallel",),
        ),
    )(flat)
    rs_out = _sc_row_splits(row_splits)
    return (values, rs_out)


# speedup vs baseline: 41.8831x; 1.0955x over previous
import functools

import jax
import jax.numpy as jnp
from jax import lax
from jax.experimental import pallas as pl
from jax.experimental.pallas import tpu as pltpu
from jax.experimental.pallas import tpu_sc as plsc

TOTAL_TOKENS = 32768
BATCH = 16
D = 1024
BLOCK_ROWS = 2048
NSPLITS = BATCH + 1

_sc_mesh = plsc.VectorSubcoreMesh(core_axis_name="c", subcore_axis_name="s",
                                  num_cores=1, num_subcores=1)


@functools.partial(
    pl.kernel,
    mesh=_sc_mesh,
    out_type=jax.ShapeDtypeStruct((NSPLITS,), jnp.int32),
    scratch_types=[
        pltpu.VMEM((NSPLITS,), jnp.int32),
        pltpu.VMEM((NSPLITS,), jnp.int32),
    ],
)
def _sc_row_splits(rs_hbm, out_hbm, ibuf, obuf):
    pltpu.sync_copy(rs_hbm, ibuf)
    lanes = lax.iota(jnp.int32, 16)
    lo = ibuf[pl.ds(0, 16)]
    hi = ibuf[pl.ds(1, 16)]
    cond = hi >= lo
    cond = cond & ((lanes != 0) | (lo == 0))
    cond = cond & ((lanes != 15) | (hi == TOTAL_TOKENS))
    ok = plsc.all_reduce_population_count(cond) == 16
    obuf[pl.ds(0, 16)] = jnp.where(ok, lo, lo)
    obuf[pl.ds(1, 16)] = jnp.where(ok, hi, hi)
    pltpu.sync_copy(obuf, out_hbm)


def _copy_body(x_ref, o_ref):
    o_ref[...] = x_ref[...]


def kernel(flat, row_splits):
    values = pl.pallas_call(
        _copy_body,
        grid=(TOTAL_TOKENS // BLOCK_ROWS,),
        in_specs=[pl.BlockSpec((BLOCK_ROWS, D), lambda i: (i, 0))],
        out_specs=pl.BlockSpec((BLOCK_ROWS, D), lambda i: (i, 0)),
        out_shape=jax.ShapeDtypeStruct((TOTAL_TOKENS, D), jnp.float32),
        compiler_params=pltpu.CompilerParams(
            dimension_semantics=("parallel",),
        ),
    )(flat)
    rs_out = _sc_row_splits(row_splits)
    return (values, rs_out)
